# baseline (speedup 1.0000x reference)
import functools

import jax
import jax.numpy as jnp
from jax import lax
from jax.experimental import pallas as pl
from jax.experimental.pallas import tpu as pltpu
from jax.experimental.pallas import tpu_sc as plsc

_LANES = 16
_CHUNK = 256
_IDXW = 128
_NW = 32


def _vsplat(vec, lane):
    idx = jnp.full((_LANES, 1), lane, jnp.int32)
    dnums = lax.GatherDimensionNumbers(
        offset_dims=(), collapsed_slice_dims=(0,), start_index_map=(0,))
    return lax.gather(vec, idx, dnums, (1,),
                      mode=lax.GatherScatterMode.PROMISE_IN_BOUNDS)


@functools.lru_cache(maxsize=None)
def _build_sc_call(T, D, nreg, ncdr, nch, nif, ncont):
    tokens_per_worker = T // _NW
    n_chunks = tokens_per_worker // _CHUNK
    cont_base = D - ncont
    mesh = plsc.VectorSubcoreMesh(core_axis_name="c", subcore_axis_name="s")

    @functools.partial(
        pl.kernel,
        mesh=mesh,
        out_type=jax.ShapeDtypeStruct((T, D), jnp.float32),
        scratch_types=[
            pltpu.VMEM((2, _CHUNK), jnp.int32),
            pltpu.VMEM((2, _CHUNK), jnp.int32),
            pltpu.VMEM((2, _CHUNK), jnp.int32),
            pltpu.VMEM((2, _CHUNK), jnp.int32),
            pltpu.VMEM((2, _CHUNK), jnp.float32),
            pltpu.VMEM((2, _CHUNK // _IDXW, _IDXW), jnp.int32),
            pltpu.VMEM((2, _CHUNK, D), jnp.float32),
            pltpu.VMEM((ncont,), jnp.float32),
            pltpu.SemaphoreType.DMA,
            pltpu.SemaphoreType.DMA,
            pltpu.SemaphoreType.DMA,
            pltpu.SemaphoreType.DMA,
            pltpu.SemaphoreType.DMA,
            pltpu.SemaphoreType.DMA,
        ],
    )
    def sc_call(n_h, reg_h, cdr_h, ch_h, if_h, tab_h, w_h, out_h,
                reg_v, cdr_v, ch_v, if_v, n_v, cidx_v, rows_v, w_v,
                isem0, isem1, gsem0, gsem1, osem0, osem1):
        isems = (isem0, isem1)
        gsems = (gsem0, gsem1)
        osems = (osem0, osem1)
        wid = lax.axis_index("s") * 2 + lax.axis_index("c")
        base = wid * tokens_per_worker

        pltpu.sync_copy(w_h, w_v)
        w_slices = [w_v[pl.ds(k * _LANES, _LANES)]
                    for k in range(ncont // _LANES)]

        def in_pairs(g, b):
            off = base + g * _CHUNK
            s = pl.ds(off, _CHUNK)
            return [(reg_h.at[s], reg_v.at[b]),
                    (cdr_h.at[s], cdr_v.at[b]),
                    (ch_h.at[s], ch_v.at[b]),
                    (if_h.at[s], if_v.at[b]),
                    (n_h.at[s], n_v.at[b])]

        def issue_in(g, b):
            for src, dst in in_pairs(g, b):
                pltpu.async_copy(src, dst, isems[b])

        def wait_in(g, b):
            for src, dst in in_pairs(g, b):
                pltpu.make_async_copy(src, dst, isems[b]).wait()

        def compute_cidx(b):
            for i in range(_CHUNK // _LANES):
                s = pl.ds((i * _LANES) % _IDXW, _LANES)
                cidx = ((reg_v[b, pl.ds(i * _LANES, _LANES)] * ncdr
                         + cdr_v[b, pl.ds(i * _LANES, _LANES)]) * nch
                        + ch_v[b, pl.ds(i * _LANES, _LANES)]) * nif \
                    + if_v[b, pl.ds(i * _LANES, _LANES)]
                cidx_v[b, i * _LANES // _IDXW, s] = cidx

        def gather_pairs(b):
            return [(tab_h.at[cidx_v.at[b, j]],
                     rows_v.at[b].at[pl.ds(j * _IDXW, _IDXW)])
                    for j in range(_CHUNK // _IDXW)]

        def issue_gather(b):
            for src, dst in gather_pairs(b):
                pltpu.async_copy(src, dst, gsems[b])

        def wait_gather(b):
            for src, dst in gather_pairs(b):
                pltpu.make_async_copy(src, dst, gsems[b]).wait()

        def out_pair(g, b):
            off = base + g * _CHUNK
            return rows_v.at[b], out_h.at[pl.ds(off, _CHUNK)]

        def issue_out(g, b):
            src, dst = out_pair(g, b)
            pltpu.async_copy(src, dst, osems[b])

        def wait_out(g, b):
            src, dst = out_pair(g, b)
            pltpu.make_async_copy(src, dst, osems[b]).wait()

        def cont_fma(b):
            for gi in range(_CHUNK // _LANES):
                n16 = n_v[b, pl.ds(gi * _LANES, _LANES)]
                for tl in range(_LANES):
                    sp = _vsplat(n16, tl)
                    t = gi * _LANES + tl
                    for k in range(ncont // _LANES):
                        plsc.addupdate(
                            rows_v.at[b, t,
                                      pl.ds(cont_base + k * _LANES, _LANES)],
                            sp * w_slices[k])

        issue_in(0, 0)
        wait_in(0, 0)
        compute_cidx(0)
        issue_gather(0)
        issue_in(1, 1)

        def pair_body(p, carry):
            for b in range(2):
                g = p * 2 + b
                nb = b ^ 1

                @pl.when(g + 1 < n_chunks)
                def _stage():
                    wait_in(g + 1, nb)
                    compute_cidx(nb)

                    @pl.when(g >= 1)
                    def _free():
                        wait_out(g - 1, nb)

                    issue_gather(nb)

                wait_gather(b)
                cont_fma(b)
                issue_out(g, b)

                @pl.when(g + 2 < n_chunks)
                def _prefetch():
                    issue_in(g + 2, b)
            return carry

        lax.fori_loop(0, n_chunks // 2, pair_body, 0)

        wait_out(n_chunks - 2, (n_chunks - 2) % 2)
        wait_out(n_chunks - 1, (n_chunks - 1) % 2)

    return sc_call


def _combined_table(W_region, W_cdr, W_chain, W_iface, b_proj):
    nreg, ncdr, nch, nif = (W_region.shape[0], W_cdr.shape[0],
                            W_chain.shape[0], W_iface.shape[0])
    rows = nreg * ncdr * nch * nif
    ridx = jnp.arange(rows)
    f = ridx % nif
    ch = (ridx // nif) % nch
    c = (ridx // (nif * nch)) % ncdr
    r = ridx // (nif * nch * ncdr)
    bias = jnp.broadcast_to(b_proj[None, :], (rows, b_proj.shape[0]))
    return jnp.concatenate(
        [W_region[r], W_cdr[c], W_chain[ch], W_iface[f], bias], axis=1)


def kernel(n, region, cdr_type, chain, interface,
           W_region, W_cdr, W_chain, W_iface, W_proj, b_proj):
    B, L = n.shape
    ncont = W_proj.shape[0]
    D = (W_region.shape[1] + W_cdr.shape[1] + W_chain.shape[1]
         + W_iface.shape[1] + ncont)
    T = B * L
    tab = _combined_table(W_region, W_cdr, W_chain, W_iface, b_proj)
    call = _build_sc_call(T, D, W_region.shape[0], W_cdr.shape[0],
                          W_chain.shape[0], W_iface.shape[0], ncont)
    out = call(n.reshape(T), region.reshape(T), cdr_type.reshape(T),
               chain.reshape(T), interface.reshape(T), tab,
               W_proj.reshape(ncont))
    return out.reshape(B, L, D)


# speedup vs baseline: 5.2413x; 5.2413x over previous
import functools

import jax
import jax.numpy as jnp
from jax import lax
from jax.experimental import pallas as pl
from jax.experimental.pallas import tpu as pltpu
from jax.experimental.pallas import tpu_sc as plsc

_LANES = 16
_CHUNK = 256
_IDXW = 128
_NW = 32


def _vsplat(vec, lane):
    idx = jnp.full((_LANES, 1), lane, jnp.int32)
    dnums = lax.GatherDimensionNumbers(
        offset_dims=(), collapsed_slice_dims=(0,), start_index_map=(0,))
    return lax.gather(vec, idx, dnums, (1,),
                      mode=lax.GatherScatterMode.PROMISE_IN_BOUNDS)


@functools.lru_cache(maxsize=None)
def _build_sc_call(T, D, nreg, ncdr, nch, nif, ncont):
    tokens_per_worker = T // _NW
    n_chunks = tokens_per_worker // _CHUNK
    cont_base = D - ncont
    mesh = plsc.VectorSubcoreMesh(core_axis_name="c", subcore_axis_name="s")

    @functools.partial(
        pl.kernel,
        mesh=mesh,
        out_type=jax.ShapeDtypeStruct((T, D), jnp.float32),
        scratch_types=[
            pltpu.VMEM((2, _CHUNK), jnp.int32),
            pltpu.VMEM((2, _CHUNK), jnp.int32),
            pltpu.VMEM((2, _CHUNK), jnp.int32),
            pltpu.VMEM((2, _CHUNK), jnp.int32),
            pltpu.VMEM((2, _CHUNK), jnp.float32),
            pltpu.VMEM((2, _CHUNK // _IDXW, _IDXW), jnp.int32),
            pltpu.VMEM((2, _CHUNK, D), jnp.float32),
            pltpu.VMEM_SHARED((nreg * ncdr * nch * nif, D), jnp.float32),
            pltpu.VMEM((ncont,), jnp.float32),
            pltpu.SemaphoreType.DMA,
            pltpu.SemaphoreType.DMA,
            pltpu.SemaphoreType.DMA,
            pltpu.SemaphoreType.DMA,
            pltpu.SemaphoreType.DMA,
            pltpu.SemaphoreType.DMA,
        ],
    )
    def sc_call(n_h, reg_h, cdr_h, ch_h, if_h, tab_h, w_h, out_h,
                reg_v, cdr_v, ch_v, if_v, n_v, cidx_v, rows_v, tab_v, w_v,
                isem0, isem1, gsem0, gsem1, osem0, osem1):
        isems = (isem0, isem1)
        gsems = (gsem0, gsem1)
        osems = (osem0, osem1)
        wid = lax.axis_index("s") * 2 + lax.axis_index("c")
        base = wid * tokens_per_worker

        pltpu.sync_copy(w_h, w_v)

        @pl.when(lax.axis_index("s") == 0)
        def _load_table():
            pltpu.sync_copy(tab_h, tab_v)

        plsc.subcore_barrier()
        w_slices = [w_v[pl.ds(k * _LANES, _LANES)]
                    for k in range(ncont // _LANES)]

        def in_pairs(g, b):
            off = base + g * _CHUNK
            s = pl.ds(off, _CHUNK)
            return [(reg_h.at[s], reg_v.at[b]),
                    (cdr_h.at[s], cdr_v.at[b]),
                    (ch_h.at[s], ch_v.at[b]),
                    (if_h.at[s], if_v.at[b]),
                    (n_h.at[s], n_v.at[b])]

        def issue_in(g, b):
            for src, dst in in_pairs(g, b):
                pltpu.async_copy(src, dst, isems[b])

        def wait_in(g, b):
            for src, dst in in_pairs(g, b):
                pltpu.make_async_copy(src, dst, isems[b]).wait()

        def compute_cidx(b):
            for i in range(_CHUNK // _LANES):
                s = pl.ds((i * _LANES) % _IDXW, _LANES)
                cidx = ((reg_v[b, pl.ds(i * _LANES, _LANES)] * ncdr
                         + cdr_v[b, pl.ds(i * _LANES, _LANES)]) * nch
                        + ch_v[b, pl.ds(i * _LANES, _LANES)]) * nif \
                    + if_v[b, pl.ds(i * _LANES, _LANES)]
                cidx_v[b, i * _LANES // _IDXW, s] = cidx

        def gather_pairs(b):
            return [(tab_v.at[cidx_v.at[b, j]],
                     rows_v.at[b].at[pl.ds(j * _IDXW, _IDXW)])
                    for j in range(_CHUNK // _IDXW)]

        def issue_gather(b):
            for src, dst in gather_pairs(b):
                pltpu.async_copy(src, dst, gsems[b])

        def wait_gather(b):
            for src, dst in gather_pairs(b):
                pltpu.make_async_copy(src, dst, gsems[b]).wait()

        def out_pair(g, b):
            off = base + g * _CHUNK
            return rows_v.at[b], out_h.at[pl.ds(off, _CHUNK)]

        def issue_out(g, b):
            src, dst = out_pair(g, b)
            pltpu.async_copy(src, dst, osems[b])

        def wait_out(g, b):
            src, dst = out_pair(g, b)
            pltpu.make_async_copy(src, dst, osems[b]).wait()

        def cont_fma(b):
            for gi in range(_CHUNK // _LANES):
                n16 = n_v[b, pl.ds(gi * _LANES, _LANES)]
                for tl in range(_LANES):
                    sp = _vsplat(n16, tl)
                    t = gi * _LANES + tl
                    for k in range(ncont // _LANES):
                        plsc.addupdate(
                            rows_v.at[b, t,
                                      pl.ds(cont_base + k * _LANES, _LANES)],
                            sp * w_slices[k])

        issue_in(0, 0)
        wait_in(0, 0)
        compute_cidx(0)
        issue_gather(0)
        issue_in(1, 1)

        def pair_body(p, carry):
            for b in range(2):
                g = p * 2 + b
                nb = b ^ 1

                @pl.when(g + 1 < n_chunks)
                def _stage():
                    wait_in(g + 1, nb)
                    compute_cidx(nb)

                    @pl.when(g >= 1)
                    def _free():
                        wait_out(g - 1, nb)

                    issue_gather(nb)

                wait_gather(b)
                cont_fma(b)
                issue_out(g, b)

                @pl.when(g + 2 < n_chunks)
                def _prefetch():
                    issue_in(g + 2, b)
            return carry

        lax.fori_loop(0, n_chunks // 2, pair_body, 0)
        wait_out(n_chunks - 2, (n_chunks - 2) % 2)
        wait_out(n_chunks - 1, (n_chunks - 1) % 2)

    return sc_call


def _combined_table(W_region, W_cdr, W_chain, W_iface, b_proj):
    nreg, ncdr, nch, nif = (W_region.shape[0], W_cdr.shape[0],
                            W_chain.shape[0], W_iface.shape[0])
    rows = nreg * ncdr * nch * nif
    ridx = jnp.arange(rows)
    f = ridx % nif
    ch = (ridx // nif) % nch
    c = (ridx // (nif * nch)) % ncdr
    r = ridx // (nif * nch * ncdr)
    bias = jnp.broadcast_to(b_proj[None, :], (rows, b_proj.shape[0]))
    return jnp.concatenate(
        [W_region[r], W_cdr[c], W_chain[ch], W_iface[f], bias], axis=1)


def kernel(n, region, cdr_type, chain, interface,
           W_region, W_cdr, W_chain, W_iface, W_proj, b_proj):
    B, L = n.shape
    ncont = W_proj.shape[0]
    D = (W_region.shape[1] + W_cdr.shape[1] + W_chain.shape[1]
         + W_iface.shape[1] + ncont)
    T = B * L
    tab = _combined_table(W_region, W_cdr, W_chain, W_iface, b_proj)
    call = _build_sc_call(T, D, W_region.shape[0], W_cdr.shape[0],
                          W_chain.shape[0], W_iface.shape[0], ncont)
    out = call(n.reshape(T), region.reshape(T), cdr_type.reshape(T),
               chain.reshape(T), interface.reshape(T), tab,
               W_proj.reshape(ncont))
    return out.reshape(B, L, D)


# speedup vs baseline: 5.3512x; 1.0210x over previous
import functools

import jax
import jax.numpy as jnp
from jax import lax
from jax.experimental import pallas as pl
from jax.experimental.pallas import tpu as pltpu
from jax.experimental.pallas import tpu_sc as plsc

_LANES = 16
_UNIT = 128
_STAGE = 256
_NBUF = 4
_NW = 32


def _vsplat(vec, lane):
    idx = jnp.full((_LANES, 1), lane, jnp.int32)
    dnums = lax.GatherDimensionNumbers(
        offset_dims=(), collapsed_slice_dims=(0,), start_index_map=(0,))
    return lax.gather(vec, idx, dnums, (1,),
                      mode=lax.GatherScatterMode.PROMISE_IN_BOUNDS)


@functools.lru_cache(maxsize=None)
def _build_sc_call(T, D, nreg, ncdr, nch, nif, ncont):
    tokens_per_worker = T // _NW
    n_units = tokens_per_worker // _UNIT
    n_stages = tokens_per_worker // _STAGE
    cont_base = D - ncont
    n_rows = nreg * ncdr * nch * nif
    mesh = plsc.VectorSubcoreMesh(core_axis_name="c", subcore_axis_name="s")

    @functools.partial(
        pl.kernel,
        mesh=mesh,
        out_type=jax.ShapeDtypeStruct((T, D), jnp.float32),
        scratch_types=[
            pltpu.VMEM((2, _STAGE), jnp.int32),
            pltpu.VMEM((2, _STAGE), jnp.int32),
            pltpu.VMEM((2, _STAGE), jnp.int32),
            pltpu.VMEM((2, _STAGE), jnp.int32),
            pltpu.VMEM((2, _STAGE), jnp.float32),
            pltpu.VMEM((_NBUF, _UNIT), jnp.int32),
            pltpu.VMEM((_NBUF, _UNIT, D), jnp.float32),
            pltpu.VMEM_SHARED((n_rows, D), jnp.float32),
            pltpu.VMEM((ncont,), jnp.float32),
            pltpu.SemaphoreType.DMA,
            pltpu.SemaphoreType.DMA,
            pltpu.SemaphoreType.DMA,
            pltpu.SemaphoreType.DMA,
            pltpu.SemaphoreType.DMA,
            pltpu.SemaphoreType.DMA,
            pltpu.SemaphoreType.DMA,
            pltpu.SemaphoreType.DMA,
            pltpu.SemaphoreType.DMA,
            pltpu.SemaphoreType.DMA,
        ],
    )
    def sc_call(n_h, reg_h, cdr_h, ch_h, if_h, tab_h, w_h, out_h,
                reg_v, cdr_v, ch_v, if_v, n_v, cidx_v, rows_v, tab_v, w_v,
                isem0, isem1, gsem0, gsem1, gsem2, gsem3,
                osem0, osem1, osem2, osem3):
        isems = (isem0, isem1)
        gsems = (gsem0, gsem1, gsem2, gsem3)
        osems = (osem0, osem1, osem2, osem3)
        wid = lax.axis_index("s") * 2 + lax.axis_index("c")
        base = wid * tokens_per_worker

        pltpu.sync_copy(w_h, w_v)

        @pl.when(lax.axis_index("s") == 0)
        def _load_table():
            pltpu.sync_copy(tab_h, tab_v)

        plsc.subcore_barrier()
        w_slices = [w_v[pl.ds(k * _LANES, _LANES)]
                    for k in range(ncont // _LANES)]

        def in_pairs(s, p):
            sl = pl.ds(base + s * _STAGE, _STAGE)
            return [(reg_h.at[sl], reg_v.at[p]),
                    (cdr_h.at[sl], cdr_v.at[p]),
                    (ch_h.at[sl], ch_v.at[p]),
                    (if_h.at[sl], if_v.at[p]),
                    (n_h.at[sl], n_v.at[p])]

        def issue_in(s, p):
            for src, dst in in_pairs(s, p):
                pltpu.async_copy(src, dst, isems[p])

        def wait_in(s, p):
            for src, dst in in_pairs(s, p):
                pltpu.make_async_copy(src, dst, isems[p]).wait()

        def compute_cidx(r, p, half):
            for i in range(_UNIT // _LANES):
                sl = pl.ds(half * _UNIT + i * _LANES, _LANES)
                cidx = ((reg_v[p, sl] * ncdr + cdr_v[p, sl]) * nch
                        + ch_v[p, sl]) * nif + if_v[p, sl]
                cidx_v[r, pl.ds(i * _LANES, _LANES)] = cidx

        def issue_gather(r):
            pltpu.async_copy(tab_v.at[cidx_v.at[r]], rows_v.at[r], gsems[r])

        def wait_gather(r):
            pltpu.make_async_copy(
                tab_v.at[cidx_v.at[r]], rows_v.at[r], gsems[r]).wait()

        def out_pair(u, r):
            return rows_v.at[r], out_h.at[pl.ds(base + u * _UNIT, _UNIT)]

        def issue_out(u, r):
            src, dst = out_pair(u, r)
            pltpu.async_copy(src, dst, osems[r])

        def wait_out(u, r):
            src, dst = out_pair(u, r)
            pltpu.make_async_copy(src, dst, osems[r]).wait()

        def cont_fma(r, p, half):
            for gi in range(_UNIT // _LANES):
                n16 = n_v[p, pl.ds(half * _UNIT + gi * _LANES, _LANES)]
                for tl in range(_LANES):
                    sp = _vsplat(n16, tl)
                    t = gi * _LANES + tl
                    for kk in range(ncont // _LANES):
                        plsc.addupdate(
                            rows_v.at[r, t,
                                      pl.ds(cont_base + kk * _LANES, _LANES)],
                            sp * w_slices[kk])

        issue_in(0, 0)
        issue_in(1, 1)
        wait_in(0, 0)
        compute_cidx(0, 0, 0)
        issue_gather(0)

        def body(q, carry):
            for j in range(4):
                k = q * 4 + j
                r_next = (j + 1) % _NBUF
                r_cur = j
                p_next = ((j + 1) // 2) % 2
                p_cur = (j // 2) % 2

                def _stage():
                    if j == 1:
                        wait_in(2 * q + 1, 1)
                    if j == 3:
                        wait_in(2 * q + 2, 0)
                    compute_cidx(r_next, p_next, (j + 1) % 2)

                    @pl.when(k >= 3)
                    def _free():
                        wait_out(k - 3, r_next)

                    issue_gather(r_next)

                if j == 3:
                    pl.when(k + 1 < n_units)(_stage)
                else:
                    _stage()

                wait_gather(r_cur)
                cont_fma(r_cur, p_cur, j % 2)
                issue_out(k, r_cur)

                if j == 1:
                    @pl.when(2 * q + 2 < n_stages)
                    def _pf0():
                        issue_in(2 * q + 2, 0)
                if j == 3:
                    @pl.when(2 * q + 3 < n_stages)
                    def _pf1():
                        issue_in(2 * q + 3, 1)
            return carry

        lax.fori_loop(0, n_units // 4, body, 0)

        for j in range(4):
            wait_out(n_units - 4 + j, j)

    return sc_call


def _combined_table(W_region, W_cdr, W_chain, W_iface, b_proj):
    nreg, ncdr, nch, nif = (W_region.shape[0], W_cdr.shape[0],
                            W_chain.shape[0], W_iface.shape[0])
    rows = nreg * ncdr * nch * nif
    ridx = jnp.arange(rows)
    f = ridx % nif
    ch = (ridx // nif) % nch
    c = (ridx // (nif * nch)) % ncdr
    r = ridx // (nif * nch * ncdr)
    bias = jnp.broadcast_to(b_proj[None, :], (rows, b_proj.shape[0]))
    return jnp.concatenate(
        [W_region[r], W_cdr[c], W_chain[ch], W_iface[f], bias], axis=1)


def kernel(n, region, cdr_type, chain, interface,
           W_region, W_cdr, W_chain, W_iface, W_proj, b_proj):
    B, L = n.shape
    ncont = W_proj.shape[0]
    D = (W_region.shape[1] + W_cdr.shape[1] + W_chain.shape[1]
         + W_iface.shape[1] + ncont)
    T = B * L
    tab = _combined_table(W_region, W_cdr, W_chain, W_iface, b_proj)
    call = _build_sc_call(T, D, W_region.shape[0], W_cdr.shape[0],
                          W_chain.shape[0], W_iface.shape[0], ncont)
    out = call(n.reshape(T), region.reshape(T), cdr_type.reshape(T),
               chain.reshape(T), interface.reshape(T), tab,
               W_proj.reshape(ncont))
    return out.reshape(B, L, D)
